# baseline (speedup 1.0000x reference)
import functools

import jax
import jax.numpy as jnp
from jax import lax
from jax.experimental import pallas as pl
from jax.experimental.pallas import tpu as pltpu
from jax.experimental.pallas import tpu_sc as plsc

N = 1024
D = 512
E = 16
BN = 16384
BE = 65536

NC = 2
NS = 16
NW = NC * NS
L = 16

BN_W = BN // NW
NF_CHUNK = 64
NF_STEPS = BN_W // NF_CHUNK

BN_S = BN // NS
ROWS_W = N // NW

BE_W = BE // NW
EF_CHUNK = 512
EF_STEPS = BE_W // EF_CHUNK


def _body(nodes_h, edges_h, updates_h, nidx_h, src_h, tgt_h,
          nf_out, ef_out, nn_out,
          lp_idx, lp_local, lp_shared, merge_v, g_ref, t_ref,
          nbuf, ubuf, nf_idx, nf_buf, sbuf, tbuf, fbuf, ebuf,
          sem0, sem1):
    c = lax.axis_index("c")
    s = lax.axis_index("s")
    wid = c * NS + s

    iota = lax.iota(jnp.int32, L)
    masks = [iota == l for l in range(L)]

    pltpu.sync_copy(nidx_h.at[pl.ds(s * BN_S, BN_S)], lp_idx)
    neg1 = jnp.full((L,), -1, jnp.int32)
    for i in range(BN_S // L):
        lp_local[pl.ds(i * L, L)] = neg1

    base_j = s * BN_S

    def scan_step(i, carry):
        idxv = lp_idx[pl.ds(i * L, L)]
        jv = jnp.full((L,), base_j, jnp.int32) + i * L + iota
        for l in range(L):
            plsc.store_scatter(lp_local, [idxv], jv, mask=masks[l])
        return carry

    lax.fori_loop(0, BN_S // L, scan_step, 0)

    pltpu.sync_copy(lp_local, lp_shared.at[pl.ds(s * N, N)])
    plsc.subcore_barrier()

    col0 = wid * ROWS_W
    for r in range(NS):
        pltpu.sync_copy(lp_shared.at[pl.ds(r * N + col0, ROWS_W)],
                        merge_v.at[pl.ds(r * ROWS_W, ROWS_W)])

    acc0 = neg1
    acc1 = neg1
    for r in range(NS):
        acc0 = jnp.maximum(acc0, merge_v[pl.ds(r * ROWS_W, L)])
        acc1 = jnp.maximum(acc1, merge_v[pl.ds(r * ROWS_W + L, L)])

    zero = jnp.zeros((L,), jnp.int32)
    g0 = jnp.maximum(acc0, zero)
    g1 = jnp.maximum(acc1, zero)
    dummy = jnp.full((L,), N + wid, jnp.int32)
    n0 = jnp.full((L,), col0, jnp.int32) + iota
    n1 = jnp.full((L,), col0 + L, jnp.int32) + iota
    t0 = jnp.where(acc0 >= 0, n0, dummy)
    t1 = jnp.where(acc1 >= 0, n1, dummy)
    g_ref[pl.ds(0, L)] = g0
    g_ref[pl.ds(L, L)] = g1
    t_ref[pl.ds(0, L)] = t0
    t_ref[pl.ds(L, L)] = t1

    pltpu.sync_copy(nodes_h.at[pl.ds(col0, ROWS_W)], nbuf)
    pltpu.sync_copy(nbuf, nn_out.at[pl.ds(col0, ROWS_W)])
    pltpu.async_copy(updates_h.at[g_ref], ubuf, sem0).wait()
    pltpu.async_copy(ubuf, nn_out.at[t_ref], sem0).wait()

    nf0 = wid * BN_W
    pltpu.sync_copy(nidx_h.at[pl.ds(nf0, BN_W)], nf_idx)

    def nf_gather(g, b):
        return pltpu.async_copy(
            nodes_h.at[nf_idx.at[pl.ds(g * NF_CHUNK, NF_CHUNK)]],
            nf_buf.at[b], sem0 if b == 0 else sem1)

    d = nf_gather(0, 0)
    for g in range(NF_STEPS):
        b = g % 2
        d.wait()
        if g + 1 < NF_STEPS:
            d = nf_gather(g + 1, 1 - b)
        pltpu.sync_copy(nf_buf.at[b], nf_out.at[pl.ds(nf0 + g * NF_CHUNK, NF_CHUNK)])

    for t in range(EF_STEPS):
        base = wid * BE_W + t * EF_CHUNK
        pltpu.sync_copy(src_h.at[pl.ds(base, EF_CHUNK)], sbuf)
        pltpu.sync_copy(tgt_h.at[pl.ds(base, EF_CHUNK)], tbuf)

        def flat_step(i, carry):
            fbuf[pl.ds(i * L, L)] = sbuf[pl.ds(i * L, L)] * N + tbuf[pl.ds(i * L, L)]
            return carry

        lax.fori_loop(0, EF_CHUNK // L, flat_step, 0)
        pltpu.async_copy(edges_h.at[fbuf], ebuf, sem0).wait()
        pltpu.sync_copy(ebuf, ef_out.at[pl.ds(base, EF_CHUNK)])


@jax.jit
def _run(nodes, edges_flat, updates, node_indices, src_indices, tgt_indices):
    mesh = plsc.VectorSubcoreMesh(core_axis_name="c", subcore_axis_name="s")
    f = pl.kernel(
        _body,
        out_type=(
            jax.ShapeDtypeStruct((BN, D), jnp.float32),
            jax.ShapeDtypeStruct((BE, E), jnp.float32),
            jax.ShapeDtypeStruct((N + NW, D), jnp.float32),
        ),
        mesh=mesh,
        compiler_params=pltpu.CompilerParams(
            needs_layout_passes=False,
            use_tc_tiling_on_sc=False,
        ),
        scratch_types=[
            pltpu.VMEM((BN_S,), jnp.int32),
            pltpu.VMEM((N,), jnp.int32),
            pltpu.VMEM_SHARED((NS * N,), jnp.int32),
            pltpu.VMEM((NS * ROWS_W,), jnp.int32),
            pltpu.VMEM((ROWS_W,), jnp.int32),
            pltpu.VMEM((ROWS_W,), jnp.int32),
            pltpu.VMEM((ROWS_W, D), jnp.float32),
            pltpu.VMEM((ROWS_W, D), jnp.float32),
            pltpu.VMEM((BN_W,), jnp.int32),
            pltpu.VMEM((2, NF_CHUNK, D), jnp.float32),
            pltpu.VMEM((EF_CHUNK,), jnp.int32),
            pltpu.VMEM((EF_CHUNK,), jnp.int32),
            pltpu.VMEM((EF_CHUNK,), jnp.int32),
            pltpu.VMEM((EF_CHUNK, E), jnp.float32),
            pltpu.SemaphoreType.DMA,
            pltpu.SemaphoreType.DMA,
        ],
    )
    return f(nodes, edges_flat, updates, node_indices, src_indices, tgt_indices)


def kernel(nodes, edges, updates, node_indices, src_indices, tgt_indices):
    edges_flat = edges.reshape(N * N, E)
    nf, ef, nn_ext = _run(nodes, edges_flat, updates,
                          node_indices.astype(jnp.int32),
                          src_indices.astype(jnp.int32),
                          tgt_indices.astype(jnp.int32))
    return (nf, ef, nn_ext[:N])
